# initial kernel scaffold (unmeasured)
import jax
import jax.numpy as jnp
from jax import lax
from jax.experimental import pallas as pl
from jax.experimental.pallas import tpu as pltpu

N_DEV = 4


def kernel(x, w_mat):
    m_per, k = x.shape
    _, n = w_mat.shape
    n_per = n // N_DEV

    def body(x_ref, w_ref, out_ref, y_ref, q_ref, amax_ref, amax_recv_ref,
             data_recv_ref, amax_send_sems, amax_recv_sems, data_send_sems,
             data_recv_sems):
        my = lax.axis_index("i")

        barrier = pltpu.get_barrier_semaphore()
        for d in range(1, N_DEV):
            peer = (my + d) % N_DEV
            pl.semaphore_signal(barrier, inc=1, device_id=(peer,),
                                device_id_type=pl.DeviceIdType.MESH)
        pl.semaphore_wait(barrier, N_DEV - 1)

        y_ref[...] = jnp.dot(x_ref[...].astype(jnp.bfloat16),
                             w_ref[...].astype(jnp.bfloat16),
                             preferred_element_type=jnp.float32)
        amax = jnp.max(jnp.abs(y_ref[...]))
        amax_ref[...] = jnp.full((8, 128), amax, jnp.float32)

        amax_sends = []
        for d in range(1, N_DEV):
            peer = (my + d) % N_DEV
            slot = N_DEV - d - 1
            rdma = pltpu.make_async_remote_copy(
                src_ref=amax_ref,
                dst_ref=amax_recv_ref.at[slot],
                send_sem=amax_send_sems.at[d - 1],
                recv_sem=amax_recv_sems.at[slot],
                device_id=(peer,),
                device_id_type=pl.DeviceIdType.MESH,
            )
            rdma.start()
            amax_sends.append(rdma)
        for j in range(1, N_DEV):
            slot = j - 1
            recv = pltpu.make_async_remote_copy(
                src_ref=amax_ref,
                dst_ref=amax_recv_ref.at[slot],
                send_sem=amax_send_sems.at[0],
                recv_sem=amax_recv_sems.at[slot],
                device_id=(my,),
                device_id_type=pl.DeviceIdType.MESH,
            )
            recv.wait_recv()

        g_amax = jnp.maximum(amax, jnp.max(amax_recv_ref[...]))
        scale = g_amax / 448.0

        q_ref[...] = jnp.clip(y_ref[...] / scale, -448.0, 448.0).astype(
            jnp.float8_e4m3fn)

        out_ref[pl.ds(my * m_per, m_per), :] = (
            q_ref[:, pl.ds(my * n_per, n_per)].astype(jnp.float32) * scale)

        data_sends = []
        for d in range(1, N_DEV):
            peer = (my + d) % N_DEV
            slot = N_DEV - d - 1
            rdma = pltpu.make_async_remote_copy(
                src_ref=q_ref.at[:, pl.ds(peer * n_per, n_per)],
                dst_ref=data_recv_ref.at[slot],
                send_sem=data_send_sems.at[d - 1],
                recv_sem=data_recv_sems.at[slot],
                device_id=(peer,),
                device_id_type=pl.DeviceIdType.MESH,
            )
            rdma.start()
            data_sends.append(rdma)
        for j in range(1, N_DEV):
            slot = j - 1
            origin = (my + j) % N_DEV
            recv = pltpu.make_async_remote_copy(
                src_ref=q_ref.at[:, pl.ds(0, n_per)],
                dst_ref=data_recv_ref.at[slot],
                send_sem=data_send_sems.at[0],
                recv_sem=data_recv_sems.at[slot],
                device_id=(my,),
                device_id_type=pl.DeviceIdType.MESH,
            )
            recv.wait_recv()
            out_ref[pl.ds(origin * m_per, m_per), :] = (
                data_recv_ref[slot].astype(jnp.float32) * scale)

        for rdma in amax_sends + data_sends:
            rdma.wait_send()

    return pl.pallas_call(
        body,
        out_shape=jax.ShapeDtypeStruct((N_DEV * m_per, n_per), jnp.float32),
        in_specs=[
            pl.BlockSpec(memory_space=pltpu.VMEM),
            pl.BlockSpec(memory_space=pltpu.VMEM),
        ],
        out_specs=pl.BlockSpec(memory_space=pltpu.VMEM),
        scratch_shapes=[
            pltpu.VMEM((m_per, n), jnp.float32),
            pltpu.VMEM((m_per, n), jnp.float8_e4m3fn),
            pltpu.VMEM((8, 128), jnp.float32),
            pltpu.VMEM((N_DEV - 1, 8, 128), jnp.float32),
            pltpu.VMEM((N_DEV - 1, m_per, n_per), jnp.float8_e4m3fn),
            pltpu.SemaphoreType.DMA((N_DEV - 1,)),
            pltpu.SemaphoreType.DMA((N_DEV - 1,)),
            pltpu.SemaphoreType.DMA((N_DEV - 1,)),
            pltpu.SemaphoreType.DMA((N_DEV - 1,)),
        ],
        compiler_params=pltpu.CompilerParams(collective_id=0),
    )(x, w_mat)


# baseline (device time: 58800 ns/iter reference)
import functools

import jax
import jax.numpy as jnp
from jax import lax
from jax.experimental import pallas as pl
from jax.experimental.pallas import tpu as pltpu

N_DEV = 4
K_CHUNK = 512


def kernel(x, w_mat):
    m_per, k = x.shape
    _, n = w_mat.shape
    n_per = n // N_DEV
    n_k = k // K_CHUNK

    def body(x_ref, w_ref, out_ref, y_ref, q_ref, amax_ref, amax_recv_ref,
             data_recv_ref, amax_send_sems, amax_recv_sems, data_send_sems,
             data_recv_sems):
        my = lax.axis_index("i")
        step = pl.program_id(0)

        @pl.when(step == 0)
        def _entry_barrier():
            barrier = pltpu.get_barrier_semaphore()
            for d in range(1, N_DEV):
                peer = (my + d) % N_DEV
                pl.semaphore_signal(barrier, inc=1, device_id=(peer,),
                                    device_id_type=pl.DeviceIdType.MESH)
            pl.semaphore_wait(barrier, N_DEV - 1)

        part = jnp.dot(x_ref[...].astype(jnp.bfloat16),
                       w_ref[...].astype(jnp.bfloat16),
                       preferred_element_type=jnp.float32)

        @pl.when(step == 0)
        def _init():
            y_ref[...] = part

        @pl.when(step != 0)
        def _acc():
            y_ref[...] += part

        @pl.when(step == n_k - 1)
        def _comm():
            amax = jnp.max(jnp.abs(y_ref[...]))
            amax_ref[...] = jnp.full((8, 128), amax, jnp.float32)

            amax_sends = []
            for d in range(1, N_DEV):
                peer = (my + d) % N_DEV
                slot = N_DEV - d - 1
                rdma = pltpu.make_async_remote_copy(
                    src_ref=amax_ref,
                    dst_ref=amax_recv_ref.at[slot],
                    send_sem=amax_send_sems.at[d - 1],
                    recv_sem=amax_recv_sems.at[slot],
                    device_id=(peer,),
                    device_id_type=pl.DeviceIdType.MESH,
                )
                rdma.start()
                amax_sends.append(rdma)
            for j in range(1, N_DEV):
                slot = j - 1
                recv = pltpu.make_async_remote_copy(
                    src_ref=amax_ref,
                    dst_ref=amax_recv_ref.at[slot],
                    send_sem=amax_send_sems.at[0],
                    recv_sem=amax_recv_sems.at[slot],
                    device_id=(my,),
                    device_id_type=pl.DeviceIdType.MESH,
                )
                recv.wait_recv()

            g_amax = jnp.maximum(amax, jnp.max(amax_recv_ref[...]))
            scale = g_amax / 448.0

            q_ref[...] = jnp.clip(y_ref[...] / scale, -448.0, 448.0).astype(
                jnp.float8_e4m3fn)

            data_sends = []
            for d in range(1, N_DEV):
                peer = (my + d) % N_DEV
                slot = N_DEV - d - 1
                rdma = pltpu.make_async_remote_copy(
                    src_ref=q_ref.at[:, pl.ds(peer * n_per, n_per)],
                    dst_ref=data_recv_ref.at[slot],
                    send_sem=data_send_sems.at[d - 1],
                    recv_sem=data_recv_sems.at[slot],
                    device_id=(peer,),
                    device_id_type=pl.DeviceIdType.MESH,
                )
                rdma.start()
                data_sends.append(rdma)

            out_ref[pl.ds(my * m_per, m_per), :] = (
                q_ref[:, pl.ds(my * n_per, n_per)].astype(jnp.float32)
                * scale)

            for j in range(1, N_DEV):
                slot = j - 1
                origin = (my + j) % N_DEV
                recv = pltpu.make_async_remote_copy(
                    src_ref=q_ref.at[:, pl.ds(0, n_per)],
                    dst_ref=data_recv_ref.at[slot],
                    send_sem=data_send_sems.at[0],
                    recv_sem=data_recv_sems.at[slot],
                    device_id=(my,),
                    device_id_type=pl.DeviceIdType.MESH,
                )
                recv.wait_recv()
                out_ref[pl.ds(origin * m_per, m_per), :] = (
                    data_recv_ref[slot].astype(jnp.float32) * scale)

            for rdma in amax_sends + data_sends:
                rdma.wait_send()

    return pl.pallas_call(
        body,
        grid=(n_k,),
        out_shape=jax.ShapeDtypeStruct((N_DEV * m_per, n_per), jnp.float32),
        in_specs=[
            pl.BlockSpec((m_per, K_CHUNK), lambda s: (0, s),
                         memory_space=pltpu.VMEM),
            pl.BlockSpec((K_CHUNK, n), lambda s: (s, 0),
                         memory_space=pltpu.VMEM),
        ],
        out_specs=pl.BlockSpec((N_DEV * m_per, n_per), lambda s: (0, 0),
                               memory_space=pltpu.VMEM),
        scratch_shapes=[
            pltpu.VMEM((m_per, n), jnp.float32),
            pltpu.VMEM((m_per, n), jnp.float8_e4m3fn),
            pltpu.VMEM((8, 128), jnp.float32),
            pltpu.VMEM((N_DEV - 1, 8, 128), jnp.float32),
            pltpu.VMEM((N_DEV - 1, m_per, n_per), jnp.float8_e4m3fn),
            pltpu.SemaphoreType.DMA((N_DEV - 1,)),
            pltpu.SemaphoreType.DMA((N_DEV - 1,)),
            pltpu.SemaphoreType.DMA((N_DEV - 1,)),
            pltpu.SemaphoreType.DMA((N_DEV - 1,)),
        ],
        compiler_params=pltpu.CompilerParams(
            collective_id=0,
            dimension_semantics=("arbitrary",),
            vmem_limit_bytes=60 * 1024 * 1024,
        ),
    )(x, w_mat)


# device time: 53957 ns/iter; 1.0898x vs baseline; 1.0898x over previous
import jax
import jax.numpy as jnp
from jax import lax
from jax.experimental import pallas as pl
from jax.experimental.pallas import tpu as pltpu

N_DEV = 4
K_CHUNK = 1024


def kernel(x, w_mat):
    m_per, k = x.shape
    _, n = w_mat.shape
    n_per = n // N_DEV
    n_k = k // K_CHUNK

    def body(x_ref, w_ref, out_ref, y_ref, send_buf_ref, amax_ref,
             amax_recv_ref, data_recv_ref, amax_send_sems, amax_recv_sems,
             data_send_sems, data_recv_sems):
        my = lax.axis_index("i")
        step = pl.program_id(0)

        @pl.when(step == 0)
        def _entry_barrier():
            barrier = pltpu.get_barrier_semaphore()
            for d in range(1, N_DEV):
                peer = (my + d) % N_DEV
                pl.semaphore_signal(barrier, inc=1, device_id=(peer,),
                                    device_id_type=pl.DeviceIdType.MESH)
            pl.semaphore_wait(barrier, N_DEV - 1)

        part = jnp.dot(x_ref[...].astype(jnp.bfloat16),
                       w_ref[...].astype(jnp.bfloat16),
                       preferred_element_type=jnp.float32)

        @pl.when(step == 0)
        def _init():
            y_ref[...] = part

        @pl.when(step != 0)
        def _acc():
            y_ref[...] += part

        @pl.when(step == n_k - 1)
        def _comm():
            amax = jnp.max(jnp.abs(y_ref[...]))
            amax_ref[...] = jnp.full((8, 128), amax, jnp.float32)

            amax_sends = []
            for d in range(1, N_DEV):
                peer = (my + d) % N_DEV
                slot = N_DEV - d - 1
                rdma = pltpu.make_async_remote_copy(
                    src_ref=amax_ref,
                    dst_ref=amax_recv_ref.at[slot],
                    send_sem=amax_send_sems.at[d - 1],
                    recv_sem=amax_recv_sems.at[slot],
                    device_id=(peer,),
                    device_id_type=pl.DeviceIdType.MESH,
                )
                rdma.start()
                amax_sends.append(rdma)
            for j in range(1, N_DEV):
                slot = j - 1
                recv = pltpu.make_async_remote_copy(
                    src_ref=amax_ref,
                    dst_ref=amax_recv_ref.at[slot],
                    send_sem=amax_send_sems.at[0],
                    recv_sem=amax_recv_sems.at[slot],
                    device_id=(my,),
                    device_id_type=pl.DeviceIdType.MESH,
                )
                recv.wait_recv()

            g_amax = jnp.maximum(amax, jnp.max(amax_recv_ref[...]))
            scale = g_amax / 448.0
            inv = 448.0 / g_amax

            data_sends = []
            for d in range(1, N_DEV):
                peer = (my + d) % N_DEV
                slot = N_DEV - d - 1
                send_buf_ref[d - 1] = jnp.clip(
                    y_ref[:, pl.ds(peer * n_per, n_per)] * inv,
                    -448.0, 448.0).astype(jnp.float8_e4m3fn)
                rdma = pltpu.make_async_remote_copy(
                    src_ref=send_buf_ref.at[d - 1],
                    dst_ref=data_recv_ref.at[slot],
                    send_sem=data_send_sems.at[d - 1],
                    recv_sem=data_recv_sems.at[slot],
                    device_id=(peer,),
                    device_id_type=pl.DeviceIdType.MESH,
                )
                rdma.start()
                data_sends.append(rdma)

            q_own = jnp.clip(y_ref[:, pl.ds(my * n_per, n_per)] * inv,
                             -448.0, 448.0).astype(jnp.float8_e4m3fn)
            out_ref[pl.ds(my * m_per, m_per), :] = (
                q_own.astype(jnp.float32) * scale).astype(jnp.bfloat16)

            for j in range(1, N_DEV):
                slot = j - 1
                origin = (my + j) % N_DEV
                recv = pltpu.make_async_remote_copy(
                    src_ref=send_buf_ref.at[0],
                    dst_ref=data_recv_ref.at[slot],
                    send_sem=data_send_sems.at[0],
                    recv_sem=data_recv_sems.at[slot],
                    device_id=(my,),
                    device_id_type=pl.DeviceIdType.MESH,
                )
                recv.wait_recv()
                out_ref[pl.ds(origin * m_per, m_per), :] = (
                    data_recv_ref[slot].astype(jnp.float32)
                    * scale).astype(jnp.bfloat16)

            for rdma in amax_sends + data_sends:
                rdma.wait_send()

    return pl.pallas_call(
        body,
        grid=(n_k,),
        out_shape=jax.ShapeDtypeStruct((N_DEV * m_per, n_per), jnp.bfloat16),
        in_specs=[
            pl.BlockSpec((m_per, K_CHUNK), lambda s: (0, s),
                         memory_space=pltpu.VMEM),
            pl.BlockSpec((K_CHUNK, n), lambda s: (s, 0),
                         memory_space=pltpu.VMEM),
        ],
        out_specs=pl.BlockSpec((N_DEV * m_per, n_per), lambda s: (0, 0),
                               memory_space=pltpu.VMEM),
        scratch_shapes=[
            pltpu.VMEM((m_per, n), jnp.float32),
            pltpu.VMEM((N_DEV - 1, m_per, n_per), jnp.float8_e4m3fn),
            pltpu.VMEM((8, 128), jnp.float32),
            pltpu.VMEM((N_DEV - 1, 8, 128), jnp.float32),
            pltpu.VMEM((N_DEV - 1, m_per, n_per), jnp.float8_e4m3fn),
            pltpu.SemaphoreType.DMA((N_DEV - 1,)),
            pltpu.SemaphoreType.DMA((N_DEV - 1,)),
            pltpu.SemaphoreType.DMA((N_DEV - 1,)),
            pltpu.SemaphoreType.DMA((N_DEV - 1,)),
        ],
        compiler_params=pltpu.CompilerParams(
            collective_id=0,
            dimension_semantics=("arbitrary",),
            vmem_limit_bytes=60 * 1024 * 1024,
        ),
    )(x, w_mat)


# device time: 53740 ns/iter; 1.0942x vs baseline; 1.0040x over previous
import jax
import jax.numpy as jnp
from jax import lax
from jax.experimental import pallas as pl
from jax.experimental.pallas import tpu as pltpu

N_DEV = 4
K_CHUNK = 1024


def kernel(x, w_mat):
    m_per, k = x.shape
    _, n = w_mat.shape
    n_per = n // N_DEV
    n_k = k // K_CHUNK

    def body(x_ref, w_ref, out_ref, y_ref, send_buf_ref, amax_ref,
             amax_recv_ref, data_recv_ref, amax_send_sems, amax_recv_sems,
             data_send_sems, data_recv_sems):
        my = lax.axis_index("i")
        step = pl.program_id(0)

        part = jnp.dot(x_ref[...].astype(jnp.bfloat16),
                       w_ref[...].astype(jnp.bfloat16),
                       preferred_element_type=jnp.float32)

        @pl.when(step == 0)
        def _init():
            y_ref[...] = part

        @pl.when(step != 0)
        def _acc():
            y_ref[...] += part

        @pl.when(step == n_k - 1)
        def _comm():
            amax = jnp.max(jnp.abs(y_ref[...]))
            amax_ref[...] = jnp.full((8, 128), amax, jnp.float32)

            barrier = pltpu.get_barrier_semaphore()
            for d in range(1, N_DEV):
                peer = (my + d) % N_DEV
                pl.semaphore_signal(barrier, inc=1, device_id=(peer,),
                                    device_id_type=pl.DeviceIdType.MESH)
            pl.semaphore_wait(barrier, N_DEV - 1)

            amax_sends = []
            for d in range(1, N_DEV):
                peer = (my + d) % N_DEV
                slot = N_DEV - d - 1
                rdma = pltpu.make_async_remote_copy(
                    src_ref=amax_ref,
                    dst_ref=amax_recv_ref.at[slot],
                    send_sem=amax_send_sems.at[d - 1],
                    recv_sem=amax_recv_sems.at[slot],
                    device_id=(peer,),
                    device_id_type=pl.DeviceIdType.MESH,
                )
                rdma.start()
                amax_sends.append(rdma)
            for j in range(1, N_DEV):
                slot = j - 1
                recv = pltpu.make_async_remote_copy(
                    src_ref=amax_ref,
                    dst_ref=amax_recv_ref.at[slot],
                    send_sem=amax_send_sems.at[0],
                    recv_sem=amax_recv_sems.at[slot],
                    device_id=(my,),
                    device_id_type=pl.DeviceIdType.MESH,
                )
                recv.wait_recv()

            g_amax = jnp.maximum(amax, jnp.max(amax_recv_ref[...]))
            scale = g_amax / 448.0
            inv = 448.0 / g_amax

            data_sends = []
            for d in (2, 1, 3):
                peer = (my + d) % N_DEV
                slot = N_DEV - d - 1
                send_buf_ref[d - 1] = jnp.clip(
                    y_ref[:, pl.ds(peer * n_per, n_per)] * inv,
                    -448.0, 448.0).astype(jnp.float8_e4m3fn)
                rdma = pltpu.make_async_remote_copy(
                    src_ref=send_buf_ref.at[d - 1],
                    dst_ref=data_recv_ref.at[slot],
                    send_sem=data_send_sems.at[d - 1],
                    recv_sem=data_recv_sems.at[slot],
                    device_id=(peer,),
                    device_id_type=pl.DeviceIdType.MESH,
                )
                rdma.start()
                data_sends.append(rdma)

            q_own = jnp.clip(y_ref[:, pl.ds(my * n_per, n_per)] * inv,
                             -448.0, 448.0).astype(jnp.float8_e4m3fn)
            out_ref[pl.ds(my * m_per, m_per), :] = (
                q_own.astype(jnp.float32) * scale).astype(jnp.bfloat16)

            for j in (1, 3, 2):
                slot = j - 1
                origin = (my + j) % N_DEV
                recv = pltpu.make_async_remote_copy(
                    src_ref=send_buf_ref.at[0],
                    dst_ref=data_recv_ref.at[slot],
                    send_sem=data_send_sems.at[0],
                    recv_sem=data_recv_sems.at[slot],
                    device_id=(my,),
                    device_id_type=pl.DeviceIdType.MESH,
                )
                recv.wait_recv()
                out_ref[pl.ds(origin * m_per, m_per), :] = (
                    data_recv_ref[slot].astype(jnp.float32)
                    * scale).astype(jnp.bfloat16)

            for rdma in amax_sends + data_sends:
                rdma.wait_send()

    return pl.pallas_call(
        body,
        grid=(n_k,),
        out_shape=jax.ShapeDtypeStruct((N_DEV * m_per, n_per), jnp.bfloat16),
        in_specs=[
            pl.BlockSpec((m_per, K_CHUNK), lambda s: (0, s),
                         memory_space=pltpu.VMEM),
            pl.BlockSpec((K_CHUNK, n), lambda s: (s, 0),
                         memory_space=pltpu.VMEM),
        ],
        out_specs=pl.BlockSpec((N_DEV * m_per, n_per), lambda s: (0, 0),
                               memory_space=pltpu.VMEM),
        scratch_shapes=[
            pltpu.VMEM((m_per, n), jnp.float32),
            pltpu.VMEM((N_DEV - 1, m_per, n_per), jnp.float8_e4m3fn),
            pltpu.VMEM((8, 128), jnp.float32),
            pltpu.VMEM((N_DEV - 1, 8, 128), jnp.float32),
            pltpu.VMEM((N_DEV - 1, m_per, n_per), jnp.float8_e4m3fn),
            pltpu.SemaphoreType.DMA((N_DEV - 1,)),
            pltpu.SemaphoreType.DMA((N_DEV - 1,)),
            pltpu.SemaphoreType.DMA((N_DEV - 1,)),
            pltpu.SemaphoreType.DMA((N_DEV - 1,)),
        ],
        compiler_params=pltpu.CompilerParams(
            collective_id=0,
            dimension_semantics=("arbitrary",),
            vmem_limit_bytes=60 * 1024 * 1024,
        ),
    )(x, w_mat)
